# baseline (speedup 1.0000x reference)
import functools

import jax
import jax.numpy as jnp
from jax import lax
from jax.experimental import pallas as pl
from jax.experimental.pallas import tpu as pltpu
from jax.experimental.pallas import tpu_sc as plsc

_HI = jax.lax.Precision.HIGHEST

_NC = 1
_NS = 16
_NW = _NC * _NS


def _make_scatter(SP, EP):
    EPW = EP // _NW
    R = EPW // 128
    NSLAB = 8
    SROWS = SP // NSLAB
    SLABW = SROWS * SP
    NZ = SLABW // _NS
    f32 = jnp.float32

    @functools.partial(
        pl.kernel,
        out_type=jax.ShapeDtypeStruct((SP * SP,), f32),
        mesh=plsc.VectorSubcoreMesh(core_axis_name="c", subcore_axis_name="s",
                                    num_cores=_NC),
        scratch_types=[
            pltpu.VMEM((EPW,), jnp.int32),
            pltpu.VMEM((EPW,), jnp.int32),
            pltpu.VMEM((R, 128), jnp.int32),
            pltpu.VMEM((R, 128), f32),
            pltpu.VMEM((NZ,), f32),
            pltpu.VMEM_SHARED((SLABW + 128,), f32),
        ],
    )
    def scatter(src_hbm, dst_hbm, out_hbm, src_v, dst_v, idx_v, val_v,
                bb_v, slab_sh):
        c = lax.axis_index("c")
        s = lax.axis_index("s")
        wid = s * _NC + c

        base = wid * EPW
        pltpu.sync_copy(src_hbm.at[pl.ds(base, EPW)], src_v)
        pltpu.sync_copy(dst_hbm.at[pl.ds(base, EPW)], dst_v)

        def slab(k, _):
            lo = k * SROWS
            hi = lo + SROWS

            def zb(i, _):
                bb_v[pl.ds(i * 16, 16)] = jnp.zeros((16,), f32)
                return 0
            lax.fori_loop(0, NZ // 16, zb, 0)
            pltpu.sync_copy(bb_v, slab_sh.at[pl.ds(s * NZ, NZ)])

            for j in range(R):
                def cv(l, _):
                    sl = pl.ds(j * 128 + l * 16, 16)
                    sv = src_v[sl]
                    dv = dst_v[sl]
                    inb = jnp.logical_and(sv >= lo, sv < hi)
                    keep = jnp.logical_and(inb, sv != dv)
                    idx_v[j, pl.ds(l * 16, 16)] = jnp.where(
                        inb, (sv - lo) * SP + dv, SLABW)
                    val_v[j, pl.ds(l * 16, 16)] = jnp.where(
                        keep, f32(1.0), f32(0.0))
                    return 0
                lax.fori_loop(0, 8, cv, 0)

            plsc.subcore_barrier()
            for j in range(R):
                pltpu.sync_copy(val_v.at[j], slab_sh.at[idx_v.at[j]],
                                add=True)
            plsc.subcore_barrier()

            pltpu.sync_copy(slab_sh.at[pl.ds(s * NZ, NZ)], bb_v)
            pltpu.sync_copy(bb_v, out_hbm.at[pl.ds(k * SLABW + s * NZ, NZ)])
            plsc.subcore_barrier()
            return 0

        lax.fori_loop(0, NSLAB, slab, 0)

    return scatter


def _norm_kernel(dt_ref, at_ref):
    dt = dt_ref[...]
    deg = jnp.sum(dt, axis=1, keepdims=True)
    dis = jnp.where(deg > 0, 1.0 / jnp.sqrt(jnp.maximum(deg, 1e-12)), 0.0)
    at_ref[...] = -(dis * dt) * jnp.transpose(dis)


def _prop_kernel(at_ref, x_ref, p_ref):
    xb = x_ref[0]
    T, CW, SP = xb.shape
    res = jnp.dot(xb.reshape(T * CW, SP), at_ref[...],
                  preferred_element_type=jnp.float32, precision=_HI)
    p_ref[0] = res.reshape(T, CW, SP)


def _rec_kernel(at_ref, x_ref, p_ref,
                w01_ref, wxg_ref, whh_ref, fc_ref,
                bih_ref, bhh_ref, cb_ref, fb_ref,
                out_ref, h_ref, pr_ref, xct_ref, pct_ref, *, B, NK, T, HID):
    t = pl.program_id(0)
    f32 = jnp.float32
    SP = at_ref.shape[0]
    CW = x_ref.shape[2]
    pred = t >= NK

    @pl.when(t == 0)
    def _():
        h_ref[...] = jnp.zeros_like(h_ref)
        xct_ref[...] = jnp.zeros_like(xct_ref)
        pct_ref[...] = jnp.zeros_like(pct_ref)

    @pl.when(pred)
    def _():
        def fill(b, _):
            xc = (jnp.dot(fc_ref[...], h_ref[b], preferred_element_type=f32,
                          precision=_HI) + fb_ref[...])[0:1, :]
            xct_ref[pl.ds(b, 1), :] = xc
            return 0
        jax.lax.fori_loop(0, B, fill, 0)
        pct_ref[...] = jnp.dot(xct_ref[...], at_ref[...],
                               preferred_element_type=f32, precision=_HI)

    sub_x = jax.lax.broadcasted_iota(jnp.int32, (CW, SP), 0)
    sub8 = jax.lax.broadcasted_iota(jnp.int32, (8, SP), 0)
    i_out = t - (NK - 1)
    pm = jnp.where(pred, 1.0, 0.0).astype(f32)
    H = HID

    def body(b, _):
        h = h_ref[b]
        xb = x_ref[b, 0]
        pb = p_ref[b, 0]
        xcur = xct_ref[pl.ds(b, 1), :]
        pcb = pct_ref[pl.ds(b, 1), :]
        xf = jnp.where(jnp.logical_and(sub_x == 0, pred), xcur, xb)
        xp = jnp.concatenate([xf, pb], axis=0)
        xg = jax.nn.sigmoid(
            jnp.dot(w01_ref[...], xp, preferred_element_type=f32,
                    precision=_HI)
            + w01_ref[:, CW:CW + 1] * (pm * pcb) + cb_ref[...])
        gx = jnp.concatenate([xf, xg], axis=0)
        gi = (jnp.dot(wxg_ref[...], gx, preferred_element_type=f32,
                      precision=_HI)
              + bih_ref[...])
        gh = (jnp.dot(whh_ref[...], h, preferred_element_type=f32,
                      precision=_HI)
              + bhh_ref[...])
        r = jax.nn.sigmoid(gi[:H] + gh[:H])
        z = jax.nn.sigmoid(gi[H:2 * H] + gh[H:2 * H])
        n = jnp.tanh(gi[2 * H:] + r * gh[2 * H:])
        hn = (1.0 - z) * n + z * h
        h_ref[b] = hn
        xo = (jnp.dot(fc_ref[...], hn, preferred_element_type=f32,
                      precision=_HI) + fb_ref[...])[0:1, :]
        pr_ref[b] = jnp.where(sub8 == i_out, xo, pr_ref[b])
        return 0

    jax.lax.fori_loop(0, B, body, 0)

    @pl.when(t == T - 1)
    def _():
        out_ref[...] = pr_ref[...]


def kernel(x_hist, enc_misc, dec, edge_index, cheb_W, cheb_b,
           W_ih, W_hh, b_ih, b_hh, fc_W, fc_b):
    f32 = jnp.float32
    B, HIST, S, OUT = x_hist.shape
    FM = enc_misc.shape[-1]
    PRED = dec.shape[1]
    IN = OUT + FM
    GNN = cheb_W.shape[2]
    HID = W_hh.shape[1]
    NK = HIST
    NP = PRED - 1
    T = HIST - 1 + PRED
    SP = (S + 127) // 128 * 128
    CW = 16

    features = jnp.concatenate([enc_misc, dec], axis=1)

    xk = jnp.concatenate([x_hist, features[:, 1:NK + 1]], axis=-1)
    xk = jnp.pad(xk.transpose(0, 1, 3, 2),
                 ((0, 0), (0, 0), (0, CW - IN), (0, SP - S)))
    fp = jnp.pad(features[:, NK + 1:].transpose(0, 1, 3, 2),
                 ((0, 0), (0, 0), (1, CW - FM - 1), (0, SP - S)))
    xall = jnp.concatenate([xk, fp], axis=1)

    E = edge_index.shape[1]
    EP = -(-E // (_NW * 128)) * (_NW * 128)
    epad = jnp.pad(edge_index, ((0, 0), (0, EP - E)))
    DT = _make_scatter(SP, EP)(epad[0], epad[1]).reshape(SP, SP)

    AT = pl.pallas_call(
        _norm_kernel,
        out_shape=jax.ShapeDtypeStruct((SP, SP), f32),
    )(DT)

    pall = pl.pallas_call(
        _prop_kernel,
        grid=(B,),
        in_specs=[
            pl.BlockSpec((SP, SP), lambda b: (0, 0)),
            pl.BlockSpec((1, T, CW, SP), lambda b: (b, 0, 0, 0)),
        ],
        out_specs=pl.BlockSpec((1, T, CW, SP), lambda b: (b, 0, 0, 0)),
        out_shape=jax.ShapeDtypeStruct((B, T, CW, SP), f32),
    )(AT, xall)

    W0 = jnp.pad(cheb_W[0].T, ((0, 0), (0, CW - IN)))
    W1 = jnp.pad(cheb_W[1].T, ((0, 0), (0, CW - IN)))
    w01 = jnp.concatenate([W0, W1], axis=1)
    Wx = jnp.pad(W_ih[:, :IN], ((0, 0), (0, CW - IN)))
    wxg = jnp.concatenate([Wx, W_ih[:, IN:]], axis=1)
    Whh = W_hh
    fc = jnp.pad(fc_W, ((0, 8 - OUT), (0, 0)))
    fb = jnp.pad(fc_b[:, None], ((0, 8 - OUT), (0, 0)))
    bih = b_ih[:, None]
    bhh = b_hh[:, None]
    cb = cheb_b[:, None]

    def full(shape):
        return pl.BlockSpec(shape, lambda t: (0,) * len(shape))

    rec = functools.partial(_rec_kernel, B=B, NK=NK, T=T, HID=HID)
    out = pl.pallas_call(
        rec,
        grid=(T,),
        in_specs=[
            full((SP, SP)),
            pl.BlockSpec((B, 1, CW, SP), lambda t: (0, t, 0, 0)),
            pl.BlockSpec((B, 1, CW, SP), lambda t: (0, t, 0, 0)),
            full(w01.shape), full(wxg.shape), full(Whh.shape), full(fc.shape),
            full(bih.shape), full(bhh.shape), full(cb.shape), full(fb.shape),
        ],
        out_specs=pl.BlockSpec((B, 8, SP), lambda t: (0, 0, 0)),
        out_shape=jax.ShapeDtypeStruct((B, 8, SP), f32),
        scratch_shapes=[
            pltpu.VMEM((B, HID, SP), f32),
            pltpu.VMEM((B, 8, SP), f32),
            pltpu.VMEM((B, SP), f32),
            pltpu.VMEM((B, SP), f32),
        ],
    )(AT, xall, pall, w01, wxg, Whh, fc, bih, bhh, cb, fb)

    preds = out[:, :PRED, :S]
    return preds[..., None]


# speedup vs baseline: 1.0420x; 1.0420x over previous
import functools

import jax
import jax.numpy as jnp
from jax import lax
from jax.experimental import pallas as pl
from jax.experimental.pallas import tpu as pltpu
from jax.experimental.pallas import tpu_sc as plsc

_HI = jax.lax.Precision.HIGHEST

_NC = 1
_NS = 16
_NW = _NC * _NS


def _make_scatter(SP, EP):
    EPW = EP // _NW
    R = EPW // 128
    NSLAB = 5
    SROWS = SP // NSLAB
    SLABW = SROWS * SP
    NZ = SLABW // _NS
    f32 = jnp.float32

    @functools.partial(
        pl.kernel,
        out_type=jax.ShapeDtypeStruct((SP * SP,), f32),
        mesh=plsc.VectorSubcoreMesh(core_axis_name="c", subcore_axis_name="s",
                                    num_cores=_NC),
        scratch_types=[
            pltpu.VMEM((EPW,), jnp.int32),
            pltpu.VMEM((EPW,), jnp.int32),
            pltpu.VMEM((R, 128), jnp.int32),
            pltpu.VMEM((R, 128), f32),
            pltpu.VMEM((NZ,), f32),
            pltpu.VMEM_SHARED((SLABW + 128,), f32),
        ],
    )
    def scatter(src_hbm, dst_hbm, out_hbm, src_v, dst_v, idx_v, val_v,
                bb_v, slab_sh):
        c = lax.axis_index("c")
        s = lax.axis_index("s")
        wid = s * _NC + c

        base = wid * EPW
        pltpu.sync_copy(src_hbm.at[pl.ds(base, EPW)], src_v)
        pltpu.sync_copy(dst_hbm.at[pl.ds(base, EPW)], dst_v)

        def slab(k, _):
            lo = k * SROWS
            hi = lo + SROWS

            def zb(i, _):
                bb_v[pl.ds(i * 16, 16)] = jnp.zeros((16,), f32)
                return 0
            lax.fori_loop(0, NZ // 16, zb, 0)
            pltpu.sync_copy(bb_v, slab_sh.at[pl.ds(s * NZ, NZ)])

            for j in range(R):
                def cv(l, _):
                    sl = pl.ds(j * 128 + l * 16, 16)
                    sv = src_v[sl]
                    dv = dst_v[sl]
                    inb = jnp.logical_and(sv >= lo, sv < hi)
                    keep = jnp.logical_and(inb, sv != dv)
                    idx_v[j, pl.ds(l * 16, 16)] = jnp.where(
                        inb, (sv - lo) * SP + dv, SLABW)
                    val_v[j, pl.ds(l * 16, 16)] = jnp.where(
                        keep, f32(1.0), f32(0.0))
                    return 0
                lax.fori_loop(0, 8, cv, 0)

            plsc.subcore_barrier()
            for j in range(R):
                pltpu.sync_copy(val_v.at[j], slab_sh.at[idx_v.at[j]],
                                add=True)
            plsc.subcore_barrier()

            pltpu.sync_copy(slab_sh.at[pl.ds(s * NZ, NZ)], bb_v)
            pltpu.sync_copy(bb_v, out_hbm.at[pl.ds(k * SLABW + s * NZ, NZ)])
            plsc.subcore_barrier()
            return 0

        lax.fori_loop(0, NSLAB, slab, 0)

    return scatter


def _norm_kernel(dt_ref, at_ref):
    dt = dt_ref[...]
    deg = jnp.sum(dt, axis=1, keepdims=True)
    dis = jnp.where(deg > 0, 1.0 / jnp.sqrt(jnp.maximum(deg, 1e-12)), 0.0)
    at_ref[...] = -(dis * dt) * jnp.transpose(dis)


def _prop_kernel(at_ref, x_ref, p_ref):
    xb = x_ref[0]
    T, CW, SP = xb.shape
    res = jnp.dot(xb.reshape(T * CW, SP), at_ref[...],
                  preferred_element_type=jnp.float32, precision=_HI)
    p_ref[0] = res.reshape(T, CW, SP)


def _rec_kernel(at_ref, x_ref, p_ref,
                w01_ref, wxg_ref, whh_ref, fc_ref,
                bih_ref, bhh_ref, cb_ref, fb_ref,
                out_ref, h_ref, pr_ref, xct_ref, pct_ref, *, B, NK, T, HID):
    t = pl.program_id(0)
    f32 = jnp.float32
    SP = at_ref.shape[0]
    CW = x_ref.shape[2]
    pred = t >= NK

    @pl.when(t == 0)
    def _():
        h_ref[...] = jnp.zeros_like(h_ref)
        xct_ref[...] = jnp.zeros_like(xct_ref)
        pct_ref[...] = jnp.zeros_like(pct_ref)

    @pl.when(pred)
    def _():
        def fill(b, _):
            xc = (jnp.dot(fc_ref[...], h_ref[b], preferred_element_type=f32,
                          precision=_HI) + fb_ref[...])[0:1, :]
            xct_ref[pl.ds(b, 1), :] = xc
            return 0
        jax.lax.fori_loop(0, B, fill, 0)
        pct_ref[...] = jnp.dot(xct_ref[...], at_ref[...],
                               preferred_element_type=f32, precision=_HI)

    sub_x = jax.lax.broadcasted_iota(jnp.int32, (CW, SP), 0)
    sub8 = jax.lax.broadcasted_iota(jnp.int32, (8, SP), 0)
    i_out = t - (NK - 1)
    pm = jnp.where(pred, 1.0, 0.0).astype(f32)
    H = HID

    def body(b, _):
        h = h_ref[b]
        xb = x_ref[b, 0]
        pb = p_ref[b, 0]
        xcur = xct_ref[pl.ds(b, 1), :]
        pcb = pct_ref[pl.ds(b, 1), :]
        xf = jnp.where(jnp.logical_and(sub_x == 0, pred), xcur, xb)
        xp = jnp.concatenate([xf, pb], axis=0)
        xg = jax.nn.sigmoid(
            jnp.dot(w01_ref[...], xp, preferred_element_type=f32,
                    precision=_HI)
            + w01_ref[:, CW:CW + 1] * (pm * pcb) + cb_ref[...])
        gx = jnp.concatenate([xf, xg], axis=0)
        gi = (jnp.dot(wxg_ref[...], gx, preferred_element_type=f32,
                      precision=_HI)
              + bih_ref[...])
        gh = (jnp.dot(whh_ref[...], h, preferred_element_type=f32,
                      precision=_HI)
              + bhh_ref[...])
        r = jax.nn.sigmoid(gi[:H] + gh[:H])
        z = jax.nn.sigmoid(gi[H:2 * H] + gh[H:2 * H])
        n = jnp.tanh(gi[2 * H:] + r * gh[2 * H:])
        hn = (1.0 - z) * n + z * h
        h_ref[b] = hn
        xo = (jnp.dot(fc_ref[...], hn, preferred_element_type=f32,
                      precision=_HI) + fb_ref[...])[0:1, :]
        pr_ref[b] = jnp.where(sub8 == i_out, xo, pr_ref[b])
        return 0

    jax.lax.fori_loop(0, B, body, 0)

    @pl.when(t == T - 1)
    def _():
        out_ref[...] = pr_ref[...]


def kernel(x_hist, enc_misc, dec, edge_index, cheb_W, cheb_b,
           W_ih, W_hh, b_ih, b_hh, fc_W, fc_b):
    f32 = jnp.float32
    B, HIST, S, OUT = x_hist.shape
    FM = enc_misc.shape[-1]
    PRED = dec.shape[1]
    IN = OUT + FM
    GNN = cheb_W.shape[2]
    HID = W_hh.shape[1]
    NK = HIST
    NP = PRED - 1
    T = HIST - 1 + PRED
    SP = (S + 127) // 128 * 128
    CW = 16

    features = jnp.concatenate([enc_misc, dec], axis=1)

    xk = jnp.concatenate([x_hist, features[:, 1:NK + 1]], axis=-1)
    xk = jnp.pad(xk.transpose(0, 1, 3, 2),
                 ((0, 0), (0, 0), (0, CW - IN), (0, SP - S)))
    fp = jnp.pad(features[:, NK + 1:].transpose(0, 1, 3, 2),
                 ((0, 0), (0, 0), (1, CW - FM - 1), (0, SP - S)))
    xall = jnp.concatenate([xk, fp], axis=1)

    E = edge_index.shape[1]
    EP = -(-E // (_NW * 128)) * (_NW * 128)
    epad = jnp.pad(edge_index, ((0, 0), (0, EP - E)))
    DT = _make_scatter(SP, EP)(epad[0], epad[1]).reshape(SP, SP)

    AT = pl.pallas_call(
        _norm_kernel,
        out_shape=jax.ShapeDtypeStruct((SP, SP), f32),
    )(DT)

    pall = pl.pallas_call(
        _prop_kernel,
        grid=(B,),
        in_specs=[
            pl.BlockSpec((SP, SP), lambda b: (0, 0)),
            pl.BlockSpec((1, T, CW, SP), lambda b: (b, 0, 0, 0)),
        ],
        out_specs=pl.BlockSpec((1, T, CW, SP), lambda b: (b, 0, 0, 0)),
        out_shape=jax.ShapeDtypeStruct((B, T, CW, SP), f32),
    )(AT, xall)

    W0 = jnp.pad(cheb_W[0].T, ((0, 0), (0, CW - IN)))
    W1 = jnp.pad(cheb_W[1].T, ((0, 0), (0, CW - IN)))
    w01 = jnp.concatenate([W0, W1], axis=1)
    Wx = jnp.pad(W_ih[:, :IN], ((0, 0), (0, CW - IN)))
    wxg = jnp.concatenate([Wx, W_ih[:, IN:]], axis=1)
    Whh = W_hh
    fc = jnp.pad(fc_W, ((0, 8 - OUT), (0, 0)))
    fb = jnp.pad(fc_b[:, None], ((0, 8 - OUT), (0, 0)))
    bih = b_ih[:, None]
    bhh = b_hh[:, None]
    cb = cheb_b[:, None]

    def full(shape):
        return pl.BlockSpec(shape, lambda t: (0,) * len(shape))

    rec = functools.partial(_rec_kernel, B=B, NK=NK, T=T, HID=HID)
    out = pl.pallas_call(
        rec,
        grid=(T,),
        in_specs=[
            full((SP, SP)),
            pl.BlockSpec((B, 1, CW, SP), lambda t: (0, t, 0, 0)),
            pl.BlockSpec((B, 1, CW, SP), lambda t: (0, t, 0, 0)),
            full(w01.shape), full(wxg.shape), full(Whh.shape), full(fc.shape),
            full(bih.shape), full(bhh.shape), full(cb.shape), full(fb.shape),
        ],
        out_specs=pl.BlockSpec((B, 8, SP), lambda t: (0, 0, 0)),
        out_shape=jax.ShapeDtypeStruct((B, 8, SP), f32),
        scratch_shapes=[
            pltpu.VMEM((B, HID, SP), f32),
            pltpu.VMEM((B, 8, SP), f32),
            pltpu.VMEM((B, SP), f32),
            pltpu.VMEM((B, SP), f32),
        ],
    )(AT, xall, pall, w01, wxg, Whh, fc, bih, bhh, cb, fb)

    preds = out[:, :PRED, :S]
    return preds[..., None]


# speedup vs baseline: 1.0427x; 1.0007x over previous
import functools

import jax
import jax.numpy as jnp
from jax import lax
from jax.experimental import pallas as pl
from jax.experimental.pallas import tpu as pltpu
from jax.experimental.pallas import tpu_sc as plsc

_HI = jax.lax.Precision.HIGHEST

_NC = 1
_NS = 16
_NW = _NC * _NS


def _make_scatter(SP, EP):
    EPW = EP // _NW
    R = EPW // 128
    NSLAB = 5
    SROWS = SP // NSLAB
    SLABW = SROWS * SP
    NZ = SLABW // _NS
    f32 = jnp.float32

    @functools.partial(
        pl.kernel,
        out_type=jax.ShapeDtypeStruct((SP * SP,), f32),
        mesh=plsc.VectorSubcoreMesh(core_axis_name="c", subcore_axis_name="s",
                                    num_cores=_NC),
        scratch_types=[
            pltpu.VMEM((EPW,), jnp.int32),
            pltpu.VMEM((EPW,), jnp.int32),
            pltpu.VMEM((R, 128), jnp.int32),
            pltpu.VMEM((R, 128), f32),
            pltpu.VMEM((NZ,), f32),
            pltpu.VMEM_SHARED((SLABW + 128,), f32),
        ],
    )
    def scatter(src_hbm, dst_hbm, out_hbm, src_v, dst_v, idx_v, val_v,
                bb_v, slab_sh):
        c = lax.axis_index("c")
        s = lax.axis_index("s")
        wid = s * _NC + c

        base = wid * EPW
        pltpu.sync_copy(src_hbm.at[pl.ds(base, EPW)], src_v)
        pltpu.sync_copy(dst_hbm.at[pl.ds(base, EPW)], dst_v)

        def slab(k, _):
            lo = k * SROWS
            hi = lo + SROWS

            def zb(i, _):
                bb_v[pl.ds(i * 16, 16)] = jnp.zeros((16,), f32)
                return 0
            lax.fori_loop(0, NZ // 16, zb, 0)
            pltpu.sync_copy(bb_v, slab_sh.at[pl.ds(s * NZ, NZ)])

            for j in range(R):
                def cv(l, _):
                    sl = pl.ds(j * 128 + l * 16, 16)
                    sv = src_v[sl]
                    dv = dst_v[sl]
                    inb = jnp.logical_and(sv >= lo, sv < hi)
                    keep = jnp.logical_and(inb, sv != dv)
                    idx_v[j, pl.ds(l * 16, 16)] = jnp.where(
                        inb, (sv - lo) * SP + dv, SLABW)
                    val_v[j, pl.ds(l * 16, 16)] = jnp.where(
                        keep, f32(1.0), f32(0.0))
                    return 0
                lax.fori_loop(0, 8, cv, 0)

            plsc.subcore_barrier()
            for j in range(R):
                pltpu.sync_copy(val_v.at[j], slab_sh.at[idx_v.at[j]],
                                add=True)
            plsc.subcore_barrier()

            pltpu.sync_copy(slab_sh.at[pl.ds(s * NZ, NZ)], bb_v)
            pltpu.sync_copy(bb_v, out_hbm.at[pl.ds(k * SLABW + s * NZ, NZ)])
            plsc.subcore_barrier()
            return 0

        lax.fori_loop(0, NSLAB, slab, 0)

    return scatter


def _norm_kernel(dt_ref, at_ref):
    dt = dt_ref[...]
    deg = jnp.sum(dt, axis=1, keepdims=True)
    dis = jnp.where(deg > 0, 1.0 / jnp.sqrt(jnp.maximum(deg, 1e-12)), 0.0)
    at_ref[...] = -(dis * dt) * jnp.transpose(dis)


def _prop_kernel(at_ref, x_ref, p_ref):
    xb = x_ref[0]
    T, CW, SP = xb.shape
    res = jnp.dot(xb.reshape(T * CW, SP), at_ref[...],
                  preferred_element_type=jnp.float32, precision=_HI)
    p_ref[0] = res.reshape(T, CW, SP)


def _rec_kernel(at_ref, x_ref, p_ref,
                w01_ref, wxg_ref, whh_ref, fc_ref,
                bih_ref, bhh_ref, cb_ref, fb_ref,
                out_ref, h_ref, pr_ref, xct_ref, pct_ref, *, B, NK, T, HID):
    t = pl.program_id(0)
    f32 = jnp.float32
    SP = at_ref.shape[0]
    CW = x_ref.shape[2]
    pred = t >= NK

    @pl.when(t == 0)
    def _():
        h_ref[...] = jnp.zeros_like(h_ref)
        xct_ref[...] = jnp.zeros_like(xct_ref)
        pct_ref[...] = jnp.zeros_like(pct_ref)

    @pl.when(pred)
    def _():
        def fill(b, _):
            xc = (jnp.dot(fc_ref[...], h_ref[b], preferred_element_type=f32,
                          precision=_HI) + fb_ref[...])[0:1, :]
            xct_ref[pl.ds(b, 1), :] = xc
            return 0
        jax.lax.fori_loop(0, B, fill, 0)
        pct_ref[...] = jnp.dot(xct_ref[...], at_ref[...],
                               preferred_element_type=f32, precision=_HI)

    sub_x = jax.lax.broadcasted_iota(jnp.int32, (CW, SP), 0)
    sub8 = jax.lax.broadcasted_iota(jnp.int32, (8, SP), 0)
    i_out = t - (NK - 1)
    pm = jnp.where(pred, 1.0, 0.0).astype(f32)
    H = HID

    def body(b, _):
        h = h_ref[b]
        xb = x_ref[b, 0]
        pb = p_ref[b, 0]
        xcur = xct_ref[pl.ds(b, 1), :]
        pcb = pct_ref[pl.ds(b, 1), :]
        xf = jnp.where(jnp.logical_and(sub_x == 0, pred), xcur, xb)
        xp = jnp.concatenate([xf, pb], axis=0)
        xg = jax.nn.sigmoid(
            jnp.dot(w01_ref[...], xp, preferred_element_type=f32,
                    precision=_HI)
            + w01_ref[:, CW:CW + 1] * (pm * pcb) + cb_ref[...])
        gx = jnp.concatenate([xf, xg], axis=0)
        gi = (jnp.dot(wxg_ref[...], gx, preferred_element_type=f32,
                      precision=_HI)
              + bih_ref[...])
        gh = (jnp.dot(whh_ref[...], h, preferred_element_type=f32,
                      precision=_HI)
              + bhh_ref[...])
        r = jax.nn.sigmoid(gi[:H] + gh[:H])
        z = jax.nn.sigmoid(gi[H:2 * H] + gh[H:2 * H])
        n = jnp.tanh(gi[2 * H:] + r * gh[2 * H:])
        hn = (1.0 - z) * n + z * h
        h_ref[b] = hn
        xo = (jnp.dot(fc_ref[...], hn, preferred_element_type=f32,
                      precision=_HI) + fb_ref[...])[0:1, :]
        pr_ref[b] = jnp.where(sub8 == i_out, xo, pr_ref[b])
        return 0

    jax.lax.fori_loop(0, B, body, 0)

    @pl.when(t == T - 1)
    def _():
        out_ref[...] = pr_ref[...]


def kernel(x_hist, enc_misc, dec, edge_index, cheb_W, cheb_b,
           W_ih, W_hh, b_ih, b_hh, fc_W, fc_b):
    f32 = jnp.float32
    B, HIST, S, OUT = x_hist.shape
    FM = enc_misc.shape[-1]
    PRED = dec.shape[1]
    IN = OUT + FM
    GNN = cheb_W.shape[2]
    HID = W_hh.shape[1]
    NK = HIST
    NP = PRED - 1
    T = HIST - 1 + PRED
    SP = (S + 127) // 128 * 128
    CW = 16

    features = jnp.concatenate([enc_misc, dec], axis=1)

    xk = jnp.concatenate([x_hist, features[:, 1:NK + 1]], axis=-1)
    xk = jnp.pad(xk.transpose(0, 1, 3, 2),
                 ((0, 0), (0, 0), (0, CW - IN), (0, SP - S)))
    fp = jnp.pad(features[:, NK + 1:].transpose(0, 1, 3, 2),
                 ((0, 0), (0, 0), (1, CW - FM - 1), (0, SP - S)))
    xall = jnp.concatenate([xk, fp], axis=1)

    E = edge_index.shape[1]
    EP = -(-E // (_NW * 128)) * (_NW * 128)
    epad = jnp.pad(edge_index, ((0, 0), (0, EP - E)))
    DT = _make_scatter(SP, EP)(epad[0], epad[1]).reshape(SP, SP)

    AT = pl.pallas_call(
        _norm_kernel,
        out_shape=jax.ShapeDtypeStruct((SP, SP), f32),
    )(DT)

    pall = pl.pallas_call(
        _prop_kernel,
        grid=(B,),
        in_specs=[
            pl.BlockSpec((SP, SP), lambda b: (0, 0)),
            pl.BlockSpec((1, T, CW, SP), lambda b: (b, 0, 0, 0)),
        ],
        out_specs=pl.BlockSpec((1, T, CW, SP), lambda b: (b, 0, 0, 0)),
        out_shape=jax.ShapeDtypeStruct((B, T, CW, SP), f32),
        compiler_params=pltpu.CompilerParams(
            dimension_semantics=("parallel",)),
    )(AT, xall)

    W0 = jnp.pad(cheb_W[0].T, ((0, 0), (0, CW - IN)))
    W1 = jnp.pad(cheb_W[1].T, ((0, 0), (0, CW - IN)))
    w01 = jnp.concatenate([W0, W1], axis=1)
    Wx = jnp.pad(W_ih[:, :IN], ((0, 0), (0, CW - IN)))
    wxg = jnp.concatenate([Wx, W_ih[:, IN:]], axis=1)
    Whh = W_hh
    fc = jnp.pad(fc_W, ((0, 8 - OUT), (0, 0)))
    fb = jnp.pad(fc_b[:, None], ((0, 8 - OUT), (0, 0)))
    bih = b_ih[:, None]
    bhh = b_hh[:, None]
    cb = cheb_b[:, None]

    def full(shape):
        return pl.BlockSpec(shape, lambda t: (0,) * len(shape))

    rec = functools.partial(_rec_kernel, B=B, NK=NK, T=T, HID=HID)
    out = pl.pallas_call(
        rec,
        grid=(T,),
        in_specs=[
            full((SP, SP)),
            pl.BlockSpec((B, 1, CW, SP), lambda t: (0, t, 0, 0)),
            pl.BlockSpec((B, 1, CW, SP), lambda t: (0, t, 0, 0)),
            full(w01.shape), full(wxg.shape), full(Whh.shape), full(fc.shape),
            full(bih.shape), full(bhh.shape), full(cb.shape), full(fb.shape),
        ],
        out_specs=pl.BlockSpec((B, 8, SP), lambda t: (0, 0, 0)),
        out_shape=jax.ShapeDtypeStruct((B, 8, SP), f32),
        scratch_shapes=[
            pltpu.VMEM((B, HID, SP), f32),
            pltpu.VMEM((B, 8, SP), f32),
            pltpu.VMEM((B, SP), f32),
            pltpu.VMEM((B, SP), f32),
        ],
    )(AT, xall, pall, w01, wxg, Whh, fc, bih, bhh, cb, fb)

    preds = out[:, :PRED, :S]
    return preds[..., None]


# speedup vs baseline: 1.0990x; 1.0540x over previous
import functools

import jax
import jax.numpy as jnp
from jax import lax
from jax.experimental import pallas as pl
from jax.experimental.pallas import tpu as pltpu
from jax.experimental.pallas import tpu_sc as plsc

_HI = jax.lax.Precision.HIGHEST

_NC = 1
_NS = 16
_NW = _NC * _NS


def _make_scatter(SP, EP):
    EPW = EP // _NW
    R = EPW // 128
    NSLAB = 5
    SROWS = SP // NSLAB
    SLABW = SROWS * SP
    NZ = SLABW // _NS
    f32 = jnp.float32

    @functools.partial(
        pl.kernel,
        out_type=jax.ShapeDtypeStruct((SP * SP,), f32),
        mesh=plsc.VectorSubcoreMesh(core_axis_name="c", subcore_axis_name="s",
                                    num_cores=_NC),
        scratch_types=[
            pltpu.VMEM((EPW,), jnp.int32),
            pltpu.VMEM((EPW,), jnp.int32),
            pltpu.VMEM((R, 128), jnp.int32),
            pltpu.VMEM((R, 128), f32),
            pltpu.VMEM((NZ,), f32),
            pltpu.VMEM_SHARED((SLABW + 128,), f32),
        ],
    )
    def scatter(src_hbm, dst_hbm, out_hbm, src_v, dst_v, idx_v, val_v,
                bb_v, slab_sh):
        c = lax.axis_index("c")
        s = lax.axis_index("s")
        wid = s * _NC + c

        base = wid * EPW
        pltpu.sync_copy(src_hbm.at[pl.ds(base, EPW)], src_v)
        pltpu.sync_copy(dst_hbm.at[pl.ds(base, EPW)], dst_v)

        def slab(k, _):
            lo = k * SROWS
            hi = lo + SROWS

            def zb(i, _):
                bb_v[pl.ds(i * 16, 16)] = jnp.zeros((16,), f32)
                return 0
            lax.fori_loop(0, NZ // 16, zb, 0)
            pltpu.sync_copy(bb_v, slab_sh.at[pl.ds(s * NZ, NZ)])

            for j in range(R):
                def cv(l, _):
                    sl = pl.ds(j * 128 + l * 16, 16)
                    sv = src_v[sl]
                    dv = dst_v[sl]
                    inb = jnp.logical_and(sv >= lo, sv < hi)
                    keep = jnp.logical_and(inb, sv != dv)
                    idx_v[j, pl.ds(l * 16, 16)] = jnp.where(
                        inb, (sv - lo) * SP + dv, SLABW)
                    val_v[j, pl.ds(l * 16, 16)] = jnp.where(
                        keep, f32(1.0), f32(0.0))
                    return 0
                lax.fori_loop(0, 8, cv, 0)

            plsc.subcore_barrier()
            for j in range(R):
                pltpu.sync_copy(val_v.at[j], slab_sh.at[idx_v.at[j]],
                                add=True)
            plsc.subcore_barrier()

            pltpu.sync_copy(slab_sh.at[pl.ds(s * NZ, NZ)], bb_v)
            pltpu.sync_copy(bb_v, out_hbm.at[pl.ds(k * SLABW + s * NZ, NZ)])
            plsc.subcore_barrier()
            return 0

        lax.fori_loop(0, NSLAB, slab, 0)

    return scatter


def _norm_kernel(dt_ref, at_ref, ah_ref, al_ref):
    dt = dt_ref[...]
    deg = jnp.sum(dt, axis=1, keepdims=True)
    dis = jnp.where(deg > 0, 1.0 / jnp.sqrt(jnp.maximum(deg, 1e-12)), 0.0)
    at = -(dis * dt) * jnp.transpose(dis)
    at_ref[...] = at
    ah = at.astype(jnp.bfloat16)
    ah_ref[...] = ah
    al_ref[...] = (at - ah.astype(jnp.float32)).astype(jnp.bfloat16)


def _prop_kernel(ah_ref, al_ref, x_ref, p_ref):
    xb = x_ref[0]
    T, CW, SP = xb.shape
    x = xb.reshape(T * CW, SP)
    xh = x.astype(jnp.bfloat16)
    xl = (x - xh.astype(jnp.float32)).astype(jnp.bfloat16)
    res = (jnp.dot(xh, ah_ref[...], preferred_element_type=jnp.float32)
           + jnp.dot(xh, al_ref[...], preferred_element_type=jnp.float32)
           + jnp.dot(xl, ah_ref[...], preferred_element_type=jnp.float32))
    p_ref[0] = res.reshape(T, CW, SP)


def _rec_kernel(at_ref, x_ref, p_ref,
                w01_ref, wxg_ref, whh_ref, fc_ref,
                bih_ref, bhh_ref, cb_ref, fb_ref,
                out_ref, h_ref, pr_ref, xct_ref, pct_ref, *, B, NK, T, HID):
    t = pl.program_id(0)
    f32 = jnp.float32
    SP = at_ref.shape[0]
    CW = x_ref.shape[2]
    pred = t >= NK

    @pl.when(t == 0)
    def _():
        h_ref[...] = jnp.zeros_like(h_ref)
        xct_ref[...] = jnp.zeros_like(xct_ref)
        pct_ref[...] = jnp.zeros_like(pct_ref)

    @pl.when(pred)
    def _():
        def fill(b, _):
            xc = (jnp.dot(fc_ref[...], h_ref[b], preferred_element_type=f32,
                          precision=_HI) + fb_ref[...])[0:1, :]
            xct_ref[pl.ds(b, 1), :] = xc
            return 0
        jax.lax.fori_loop(0, B, fill, 0)
        pct_ref[...] = jnp.dot(xct_ref[...], at_ref[...],
                               preferred_element_type=f32, precision=_HI)

    sub_x = jax.lax.broadcasted_iota(jnp.int32, (CW, SP), 0)
    sub8 = jax.lax.broadcasted_iota(jnp.int32, (8, SP), 0)
    i_out = t - (NK - 1)
    pm = jnp.where(pred, 1.0, 0.0).astype(f32)
    H = HID

    def body(b, _):
        h = h_ref[b]
        xb = x_ref[b, 0]
        pb = p_ref[b, 0]
        xcur = xct_ref[pl.ds(b, 1), :]
        pcb = pct_ref[pl.ds(b, 1), :]
        xf = jnp.where(jnp.logical_and(sub_x == 0, pred), xcur, xb)
        xp = jnp.concatenate([xf, pb], axis=0)
        xg = jax.nn.sigmoid(
            jnp.dot(w01_ref[...], xp, preferred_element_type=f32,
                    precision=_HI)
            + w01_ref[:, CW:CW + 1] * (pm * pcb) + cb_ref[...])
        gx = jnp.concatenate([xf, xg], axis=0)
        gi = (jnp.dot(wxg_ref[...], gx, preferred_element_type=f32,
                      precision=_HI)
              + bih_ref[...])
        gh = (jnp.dot(whh_ref[...], h, preferred_element_type=f32,
                      precision=_HI)
              + bhh_ref[...])
        r = jax.nn.sigmoid(gi[:H] + gh[:H])
        z = jax.nn.sigmoid(gi[H:2 * H] + gh[H:2 * H])
        n = jnp.tanh(gi[2 * H:] + r * gh[2 * H:])
        hn = (1.0 - z) * n + z * h
        h_ref[b] = hn
        xo = (jnp.dot(fc_ref[...], hn, preferred_element_type=f32,
                      precision=_HI) + fb_ref[...])[0:1, :]
        pr_ref[b] = jnp.where(sub8 == i_out, xo, pr_ref[b])
        return 0

    jax.lax.fori_loop(0, B, body, 0)

    @pl.when(t == T - 1)
    def _():
        out_ref[...] = pr_ref[...]


def kernel(x_hist, enc_misc, dec, edge_index, cheb_W, cheb_b,
           W_ih, W_hh, b_ih, b_hh, fc_W, fc_b):
    f32 = jnp.float32
    B, HIST, S, OUT = x_hist.shape
    FM = enc_misc.shape[-1]
    PRED = dec.shape[1]
    IN = OUT + FM
    GNN = cheb_W.shape[2]
    HID = W_hh.shape[1]
    NK = HIST
    NP = PRED - 1
    T = HIST - 1 + PRED
    SP = (S + 127) // 128 * 128
    CW = 16

    features = jnp.concatenate([enc_misc, dec], axis=1)

    xk = jnp.concatenate([x_hist, features[:, 1:NK + 1]], axis=-1)
    xk = jnp.pad(xk.transpose(0, 1, 3, 2),
                 ((0, 0), (0, 0), (0, CW - IN), (0, SP - S)))
    fp = jnp.pad(features[:, NK + 1:].transpose(0, 1, 3, 2),
                 ((0, 0), (0, 0), (1, CW - FM - 1), (0, SP - S)))
    xall = jnp.concatenate([xk, fp], axis=1)

    E = edge_index.shape[1]
    EP = -(-E // (_NW * 128)) * (_NW * 128)
    epad = jnp.pad(edge_index, ((0, 0), (0, EP - E)))
    DT = _make_scatter(SP, EP)(epad[0], epad[1]).reshape(SP, SP)

    AT, AH, AL = pl.pallas_call(
        _norm_kernel,
        out_shape=[jax.ShapeDtypeStruct((SP, SP), f32),
                   jax.ShapeDtypeStruct((SP, SP), jnp.bfloat16),
                   jax.ShapeDtypeStruct((SP, SP), jnp.bfloat16)],
    )(DT)

    pall = pl.pallas_call(
        _prop_kernel,
        grid=(B,),
        in_specs=[
            pl.BlockSpec((SP, SP), lambda b: (0, 0)),
            pl.BlockSpec((SP, SP), lambda b: (0, 0)),
            pl.BlockSpec((1, T, CW, SP), lambda b: (b, 0, 0, 0)),
        ],
        out_specs=pl.BlockSpec((1, T, CW, SP), lambda b: (b, 0, 0, 0)),
        out_shape=jax.ShapeDtypeStruct((B, T, CW, SP), f32),
        compiler_params=pltpu.CompilerParams(
            dimension_semantics=("parallel",)),
    )(AH, AL, xall)

    W0 = jnp.pad(cheb_W[0].T, ((0, 0), (0, CW - IN)))
    W1 = jnp.pad(cheb_W[1].T, ((0, 0), (0, CW - IN)))
    w01 = jnp.concatenate([W0, W1], axis=1)
    Wx = jnp.pad(W_ih[:, :IN], ((0, 0), (0, CW - IN)))
    wxg = jnp.concatenate([Wx, W_ih[:, IN:]], axis=1)
    Whh = W_hh
    fc = jnp.pad(fc_W, ((0, 8 - OUT), (0, 0)))
    fb = jnp.pad(fc_b[:, None], ((0, 8 - OUT), (0, 0)))
    bih = b_ih[:, None]
    bhh = b_hh[:, None]
    cb = cheb_b[:, None]

    def full(shape):
        return pl.BlockSpec(shape, lambda t: (0,) * len(shape))

    rec = functools.partial(_rec_kernel, B=B, NK=NK, T=T, HID=HID)
    out = pl.pallas_call(
        rec,
        grid=(T,),
        in_specs=[
            full((SP, SP)),
            pl.BlockSpec((B, 1, CW, SP), lambda t: (0, t, 0, 0)),
            pl.BlockSpec((B, 1, CW, SP), lambda t: (0, t, 0, 0)),
            full(w01.shape), full(wxg.shape), full(Whh.shape), full(fc.shape),
            full(bih.shape), full(bhh.shape), full(cb.shape), full(fb.shape),
        ],
        out_specs=pl.BlockSpec((B, 8, SP), lambda t: (0, 0, 0)),
        out_shape=jax.ShapeDtypeStruct((B, 8, SP), f32),
        scratch_shapes=[
            pltpu.VMEM((B, HID, SP), f32),
            pltpu.VMEM((B, 8, SP), f32),
            pltpu.VMEM((B, SP), f32),
            pltpu.VMEM((B, SP), f32),
        ],
    )(AT, xall, pall, w01, wxg, Whh, fc, bih, bhh, cb, fb)

    preds = out[:, :PRED, :S]
    return preds[..., None]


# speedup vs baseline: 1.4117x; 1.2844x over previous
import functools

import jax
import jax.numpy as jnp
from jax import lax
from jax.experimental import pallas as pl
from jax.experimental.pallas import tpu as pltpu
from jax.experimental.pallas import tpu_sc as plsc

_HI = jax.lax.Precision.HIGHEST

_NC = 1
_NS = 16
_NW = _NC * _NS


def _make_scatter(SP, EP):
    EPW = EP // _NW
    R = EPW // 128
    NSLAB = 5
    SROWS = SP // NSLAB
    SLABW = SROWS * SP
    NZ = SLABW // _NS
    f32 = jnp.float32

    @functools.partial(
        pl.kernel,
        out_type=jax.ShapeDtypeStruct((SP * SP,), f32),
        mesh=plsc.VectorSubcoreMesh(core_axis_name="c", subcore_axis_name="s",
                                    num_cores=_NC),
        scratch_types=[
            pltpu.VMEM((EPW,), jnp.int32),
            pltpu.VMEM((EPW,), jnp.int32),
            pltpu.VMEM((R, 128), jnp.int32),
            pltpu.VMEM((R, 128), f32),
            pltpu.VMEM((NZ,), f32),
            pltpu.VMEM_SHARED((SLABW + 128,), f32),
        ],
    )
    def scatter(src_hbm, dst_hbm, out_hbm, src_v, dst_v, idx_v, val_v,
                bb_v, slab_sh):
        c = lax.axis_index("c")
        s = lax.axis_index("s")
        wid = s * _NC + c

        base = wid * EPW
        pltpu.sync_copy(src_hbm.at[pl.ds(base, EPW)], src_v)
        pltpu.sync_copy(dst_hbm.at[pl.ds(base, EPW)], dst_v)

        def slab(k, _):
            lo = k * SROWS
            hi = lo + SROWS

            def zb(i, _):
                bb_v[pl.ds(i * 16, 16)] = jnp.zeros((16,), f32)
                return 0
            lax.fori_loop(0, NZ // 16, zb, 0)
            pltpu.sync_copy(bb_v, slab_sh.at[pl.ds(s * NZ, NZ)])

            for j in range(R):
                def cv(l, _):
                    sl = pl.ds(j * 128 + l * 16, 16)
                    sv = src_v[sl]
                    dv = dst_v[sl]
                    inb = jnp.logical_and(sv >= lo, sv < hi)
                    keep = jnp.logical_and(inb, sv != dv)
                    idx_v[j, pl.ds(l * 16, 16)] = jnp.where(
                        inb, (sv - lo) * SP + dv, SLABW)
                    val_v[j, pl.ds(l * 16, 16)] = jnp.where(
                        keep, f32(1.0), f32(0.0))
                    return 0
                lax.fori_loop(0, 8, cv, 0)

            plsc.subcore_barrier()
            for j in range(R):
                pltpu.sync_copy(val_v.at[j], slab_sh.at[idx_v.at[j]],
                                add=True)
            plsc.subcore_barrier()

            pltpu.sync_copy(slab_sh.at[pl.ds(s * NZ, NZ)], bb_v)
            pltpu.sync_copy(bb_v, out_hbm.at[pl.ds(k * SLABW + s * NZ, NZ)])
            plsc.subcore_barrier()
            return 0

        lax.fori_loop(0, NSLAB, slab, 0)

    return scatter


def _norm_kernel(dt_ref, at_ref, ah_ref, al_ref):
    dt = dt_ref[...]
    deg = jnp.sum(dt, axis=1, keepdims=True)
    dis = jnp.where(deg > 0, 1.0 / jnp.sqrt(jnp.maximum(deg, 1e-12)), 0.0)
    at = -(dis * dt) * jnp.transpose(dis)
    at_ref[...] = at
    ah = at.astype(jnp.bfloat16)
    ah_ref[...] = ah
    al_ref[...] = (at - ah.astype(jnp.float32)).astype(jnp.bfloat16)


def _prop_kernel(ah_ref, al_ref, x_ref, p_ref):
    xb = x_ref[0]
    T, CW, SP = xb.shape
    x = xb.reshape(T * CW, SP)
    xh = x.astype(jnp.bfloat16)
    xl = (x - xh.astype(jnp.float32)).astype(jnp.bfloat16)
    res = (jnp.dot(xh, ah_ref[...], preferred_element_type=jnp.float32)
           + jnp.dot(xh, al_ref[...], preferred_element_type=jnp.float32)
           + jnp.dot(xl, ah_ref[...], preferred_element_type=jnp.float32))
    p_ref[0] = res.reshape(T, CW, SP)


def _rec_kernel(at_ref, x_ref, p_ref,
                w01_ref, wxg_ref, whh_ref, fc_ref,
                bih_ref, bhh_ref, cb_ref, fb_ref,
                out_ref, h_ref, pr_ref, xct_ref, pct_ref, *, B, NK, T, HID):
    t = pl.program_id(0)
    f32 = jnp.float32
    SP = at_ref.shape[0]
    CW = x_ref.shape[2]
    pred = t >= NK

    @pl.when(t == 0)
    def _():
        h_ref[...] = jnp.zeros_like(h_ref)
        xct_ref[...] = jnp.zeros_like(xct_ref)
        pct_ref[...] = jnp.zeros_like(pct_ref)

    @pl.when(pred)
    def _():
        def fill(b, _):
            xc = (jnp.dot(fc_ref[...], h_ref[b], preferred_element_type=f32,
                          precision=_HI) + fb_ref[...])[0:1, :]
            xct_ref[pl.ds(b, 1), :] = xc
            return 0
        jax.lax.fori_loop(0, B, fill, 0)
        pct_ref[...] = jnp.dot(xct_ref[...], at_ref[...],
                               preferred_element_type=f32, precision=_HI)

    sub_x = jax.lax.broadcasted_iota(jnp.int32, (CW, SP), 0)
    sub8 = jax.lax.broadcasted_iota(jnp.int32, (8, SP), 0)
    i_out = t - (NK - 1)
    pm = jnp.where(pred, 1.0, 0.0).astype(f32)
    H = HID
    bf = jnp.bfloat16

    def split(w):
        wh = w.astype(bf)
        return wh, (w - wh.astype(f32)).astype(bf)

    w01h, w01l = split(w01_ref[...])
    wxgh, wxgl = split(wxg_ref[...])
    whhh, whhl = split(whh_ref[...])

    def d3(wh, wl, v):
        vh = v.astype(bf)
        vl = (v - vh.astype(f32)).astype(bf)
        return (jnp.dot(wh, vh, preferred_element_type=f32)
                + jnp.dot(wh, vl, preferred_element_type=f32)
                + jnp.dot(wl, vh, preferred_element_type=f32))

    def body(b, _):
        h = h_ref[b]
        xb = x_ref[b, 0]
        pb = p_ref[b, 0]
        xcur = xct_ref[pl.ds(b, 1), :]
        pcb = pct_ref[pl.ds(b, 1), :]
        xf = jnp.where(jnp.logical_and(sub_x == 0, pred), xcur, xb)
        xp = jnp.concatenate([xf, pb], axis=0)
        xg = jax.nn.sigmoid(
            d3(w01h, w01l, xp)
            + w01_ref[:, CW:CW + 1] * (pm * pcb) + cb_ref[...])
        gx = jnp.concatenate([xf, xg], axis=0)
        gi = d3(wxgh, wxgl, gx) + bih_ref[...]
        gh = d3(whhh, whhl, h) + bhh_ref[...]
        r = jax.nn.sigmoid(gi[:H] + gh[:H])
        z = jax.nn.sigmoid(gi[H:2 * H] + gh[H:2 * H])
        n = jnp.tanh(gi[2 * H:] + r * gh[2 * H:])
        hn = (1.0 - z) * n + z * h
        h_ref[b] = hn
        xo = (jnp.dot(fc_ref[...], hn, preferred_element_type=f32,
                      precision=_HI) + fb_ref[...])[0:1, :]
        pr_ref[b] = jnp.where(sub8 == i_out, xo, pr_ref[b])
        return 0

    jax.lax.fori_loop(0, B, body, 0)

    @pl.when(t == T - 1)
    def _():
        out_ref[...] = pr_ref[...]


def kernel(x_hist, enc_misc, dec, edge_index, cheb_W, cheb_b,
           W_ih, W_hh, b_ih, b_hh, fc_W, fc_b):
    f32 = jnp.float32
    B, HIST, S, OUT = x_hist.shape
    FM = enc_misc.shape[-1]
    PRED = dec.shape[1]
    IN = OUT + FM
    GNN = cheb_W.shape[2]
    HID = W_hh.shape[1]
    NK = HIST
    NP = PRED - 1
    T = HIST - 1 + PRED
    SP = (S + 127) // 128 * 128
    CW = 16

    features = jnp.concatenate([enc_misc, dec], axis=1)

    xk = jnp.concatenate([x_hist, features[:, 1:NK + 1]], axis=-1)
    xk = jnp.pad(xk.transpose(0, 1, 3, 2),
                 ((0, 0), (0, 0), (0, CW - IN), (0, SP - S)))
    fp = jnp.pad(features[:, NK + 1:].transpose(0, 1, 3, 2),
                 ((0, 0), (0, 0), (1, CW - FM - 1), (0, SP - S)))
    xall = jnp.concatenate([xk, fp], axis=1)

    E = edge_index.shape[1]
    EP = -(-E // (_NW * 128)) * (_NW * 128)
    epad = jnp.pad(edge_index, ((0, 0), (0, EP - E)))
    DT = _make_scatter(SP, EP)(epad[0], epad[1]).reshape(SP, SP)

    AT, AH, AL = pl.pallas_call(
        _norm_kernel,
        out_shape=[jax.ShapeDtypeStruct((SP, SP), f32),
                   jax.ShapeDtypeStruct((SP, SP), jnp.bfloat16),
                   jax.ShapeDtypeStruct((SP, SP), jnp.bfloat16)],
    )(DT)

    pall = pl.pallas_call(
        _prop_kernel,
        grid=(B,),
        in_specs=[
            pl.BlockSpec((SP, SP), lambda b: (0, 0)),
            pl.BlockSpec((SP, SP), lambda b: (0, 0)),
            pl.BlockSpec((1, T, CW, SP), lambda b: (b, 0, 0, 0)),
        ],
        out_specs=pl.BlockSpec((1, T, CW, SP), lambda b: (b, 0, 0, 0)),
        out_shape=jax.ShapeDtypeStruct((B, T, CW, SP), f32),
        compiler_params=pltpu.CompilerParams(
            dimension_semantics=("parallel",)),
    )(AH, AL, xall)

    W0 = jnp.pad(cheb_W[0].T, ((0, 0), (0, CW - IN)))
    W1 = jnp.pad(cheb_W[1].T, ((0, 0), (0, CW - IN)))
    w01 = jnp.concatenate([W0, W1], axis=1)
    Wx = jnp.pad(W_ih[:, :IN], ((0, 0), (0, CW - IN)))
    wxg = jnp.concatenate([Wx, W_ih[:, IN:]], axis=1)
    Whh = W_hh
    fc = jnp.pad(fc_W, ((0, 8 - OUT), (0, 0)))
    fb = jnp.pad(fc_b[:, None], ((0, 8 - OUT), (0, 0)))
    bih = b_ih[:, None]
    bhh = b_hh[:, None]
    cb = cheb_b[:, None]

    def full(shape):
        return pl.BlockSpec(shape, lambda t: (0,) * len(shape))

    rec = functools.partial(_rec_kernel, B=B, NK=NK, T=T, HID=HID)
    out = pl.pallas_call(
        rec,
        grid=(T,),
        in_specs=[
            full((SP, SP)),
            pl.BlockSpec((B, 1, CW, SP), lambda t: (0, t, 0, 0)),
            pl.BlockSpec((B, 1, CW, SP), lambda t: (0, t, 0, 0)),
            full(w01.shape), full(wxg.shape), full(Whh.shape), full(fc.shape),
            full(bih.shape), full(bhh.shape), full(cb.shape), full(fb.shape),
        ],
        out_specs=pl.BlockSpec((B, 8, SP), lambda t: (0, 0, 0)),
        out_shape=jax.ShapeDtypeStruct((B, 8, SP), f32),
        scratch_shapes=[
            pltpu.VMEM((B, HID, SP), f32),
            pltpu.VMEM((B, 8, SP), f32),
            pltpu.VMEM((B, SP), f32),
            pltpu.VMEM((B, SP), f32),
        ],
    )(AT, xall, pall, w01, wxg, Whh, fc, bih, bhh, cb, fb)

    preds = out[:, :PRED, :S]
    return preds[..., None]
